# baseline (speedup 1.0000x reference)
import functools

import jax
import jax.numpy as jnp
from jax import lax
from jax.experimental import pallas as pl
from jax.experimental.pallas import tpu as pltpu
from jax.experimental.pallas import tpu_sc as plsc

B = 16384
EMB = 32


def _sc_gather_t(utT, mtT, uidx, midx):
    info = plsc.get_sparse_core_info()
    nc, ns = info.num_cores, info.num_subcores
    nw = nc * ns
    bpw = B // nw

    mesh = plsc.VectorSubcoreMesh(core_axis_name="c", subcore_axis_name="s")

    @functools.partial(
        pl.kernel,
        mesh=mesh,
        out_type=(
            jax.ShapeDtypeStruct((EMB, B), jnp.float32),
            jax.ShapeDtypeStruct((EMB, B), jnp.float32),
        ),
        scratch_types=[
            pltpu.VMEM((bpw,), jnp.int32),
            pltpu.VMEM((bpw,), jnp.int32),
            pltpu.VMEM((EMB, bpw), jnp.float32),
            pltpu.VMEM((EMB, bpw), jnp.float32),
            pltpu.SemaphoreType.DMA,
        ],
        compiler_params=pltpu.CompilerParams(use_tc_tiling_on_sc=False),
    )
    def gather_kernel(ut_hbm, mt_hbm, uidx_hbm, midx_hbm, u_out, m_out,
                      uidx_v, midx_v, ubuf_v, mbuf_v, sem):
        wid = lax.axis_index("s") * nc + lax.axis_index("c")
        base = wid * bpw
        pltpu.sync_copy(uidx_hbm.at[pl.ds(base, bpw)], uidx_v)
        pltpu.sync_copy(midx_hbm.at[pl.ds(base, bpw)], midx_v)
        copies = []
        for c in range(EMB):
            copies.append(pltpu.async_copy(
                ut_hbm.at[c].at[uidx_v], ubuf_v.at[c], sem))
            copies.append(pltpu.async_copy(
                mt_hbm.at[c].at[midx_v], mbuf_v.at[c], sem))
        for cp in copies:
            cp.wait()
        pltpu.sync_copy(ubuf_v, u_out.at[:, pl.ds(base, bpw)])
        pltpu.sync_copy(mbuf_v, m_out.at[:, pl.ds(base, bpw)])

    return gather_kernel(utT, mtT, uidx, midx)


def _mlp_body(u_ref, m_ref, g_ref, a_ref, o_ref, genres_ref, ctab_ref,
              w1_ref, b1_ref, w2_ref, b2_ref, w3_ref, b3_ref, out_ref):
    blk = u_ref.shape[1]
    lanes = lax.broadcasted_iota(jnp.int32, (128, blk), 0)
    g = g_ref[...]
    a = a_ref[...]
    o = o_ref[...]
    oh = ((lanes == g) | (lanes == (a + 2)) | (lanes == (o + 12)))
    demo = jnp.dot(ctab_ref[...], oh.astype(jnp.float32),
                   preferred_element_type=jnp.float32)
    h1 = (
        jnp.dot(w1_ref[:, 0:32], u_ref[...], preferred_element_type=jnp.float32)
        + jnp.dot(w1_ref[:, 32:64], m_ref[...], preferred_element_type=jnp.float32)
        + jnp.dot(w1_ref[:, 64:80], demo, preferred_element_type=jnp.float32)
        + jnp.dot(w1_ref[:, 80:98], genres_ref[...], preferred_element_type=jnp.float32)
        + b1_ref[...]
    )
    h1 = jnp.maximum(h1, 0.0)
    h2 = jnp.maximum(
        jnp.dot(w2_ref[...], h1, preferred_element_type=jnp.float32) + b2_ref[...],
        0.0)
    out_ref[...] = (jnp.dot(w3_ref[...], h2, preferred_element_type=jnp.float32)
                    + b3_ref[...])


def _tc_mlp(uT, mT, g2, a2, o2, genresT, ctabT,
            W1T, b1T, W2T, b2T, W3T, b3, blk=2048):
    grid = B // blk
    full = lambda i: (0, 0)
    col = lambda i: (0, i)
    return pl.pallas_call(
        _mlp_body,
        grid=(grid,),
        in_specs=[
            pl.BlockSpec((EMB, blk), col),
            pl.BlockSpec((EMB, blk), col),
            pl.BlockSpec((1, blk), col),
            pl.BlockSpec((1, blk), col),
            pl.BlockSpec((1, blk), col),
            pl.BlockSpec((18, blk), col),
            pl.BlockSpec((16, 128), full),
            pl.BlockSpec((128, 98), full),
            pl.BlockSpec((128, 1), full),
            pl.BlockSpec((64, 128), full),
            pl.BlockSpec((64, 1), full),
            pl.BlockSpec((1, 64), full),
            pl.BlockSpec((1, 1), full),
        ],
        out_specs=pl.BlockSpec((1, blk), col),
        out_shape=jax.ShapeDtypeStruct((1, B), jnp.float32),
    )(uT, mT, g2, a2, o2, genresT, ctabT, W1T, b1T, W2T, b2T, W3T, b3)


def kernel(user, movie, gender, age, occupation, genres,
           user_table, movie_table, gender_table, age_table, occ_table,
           W1, b1, W2, b2, W3, b3):
    uidx = user.astype(jnp.int32)
    midx = movie.astype(jnp.int32)
    uT, mT = _sc_gather_t(user_table.T, movie_table.T, uidx, midx)

    ctabT = jnp.zeros((16, 128), jnp.float32)
    ctabT = ctabT.at[0:4, 0:2].set(gender_table.T)
    ctabT = ctabT.at[4:8, 2:12].set(age_table.T)
    ctabT = ctabT.at[8:16, 12:37].set(occ_table.T)

    g2 = gender.astype(jnp.int32).reshape(1, B)
    a2 = age.astype(jnp.int32).reshape(1, B)
    o2 = occupation.astype(jnp.int32).reshape(1, B)

    out = _tc_mlp(uT, mT, g2, a2, o2, genres.T, ctabT,
                  W1.T, b1.reshape(128, 1), W2.T, b2.reshape(64, 1),
                  W3.T, b3.reshape(1, 1))
    return jnp.squeeze(out, axis=0)


# speedup vs baseline: 3.8118x; 3.8118x over previous
import functools

import jax
import jax.numpy as jnp
from jax import lax
from jax.experimental import pallas as pl
from jax.experimental.pallas import tpu as pltpu
from jax.experimental.pallas import tpu_sc as plsc

B = 16384
EMB = 32
IDX_CHUNK = 128


def _sc_gather(user_table, movie_table, uidx2, midx2):
    info = plsc.get_sparse_core_info()
    nc, ns = info.num_cores, info.num_subcores
    nw = nc * ns
    bpw = B // nw
    cpw = bpw // IDX_CHUNK

    mesh = plsc.VectorSubcoreMesh(core_axis_name="c", subcore_axis_name="s")

    @functools.partial(
        pl.kernel,
        mesh=mesh,
        out_type=(
            jax.ShapeDtypeStruct((B, EMB), jnp.bfloat16),
            jax.ShapeDtypeStruct((B, EMB), jnp.bfloat16),
        ),
        scratch_types=[
            pltpu.VMEM((cpw, IDX_CHUNK), jnp.int32),
            pltpu.VMEM((cpw, IDX_CHUNK), jnp.int32),
            pltpu.VMEM((bpw, EMB), jnp.bfloat16),
            pltpu.VMEM((bpw, EMB), jnp.bfloat16),
            pltpu.SemaphoreType.DMA,
        ],
        compiler_params=pltpu.CompilerParams(use_tc_tiling_on_sc=False),
    )
    def gather_kernel(ut_hbm, mt_hbm, uidx_hbm, midx_hbm, u_out, m_out,
                      uidx_v, midx_v, urows_v, mrows_v, sem):
        wid = lax.axis_index("s") * nc + lax.axis_index("c")
        base = wid * bpw
        pltpu.sync_copy(uidx_hbm.at[pl.ds(wid * cpw, cpw)], uidx_v)
        pltpu.sync_copy(midx_hbm.at[pl.ds(wid * cpw, cpw)], midx_v)
        copies = []
        for j in range(cpw):
            copies.append(pltpu.async_copy(
                ut_hbm.at[uidx_v.at[j]],
                urows_v.at[pl.ds(j * IDX_CHUNK, IDX_CHUNK)], sem))
            copies.append(pltpu.async_copy(
                mt_hbm.at[midx_v.at[j]],
                mrows_v.at[pl.ds(j * IDX_CHUNK, IDX_CHUNK)], sem))
        for c in copies:
            c.wait()
        pltpu.sync_copy(urows_v, u_out.at[pl.ds(base, bpw)])
        pltpu.sync_copy(mrows_v, m_out.at[pl.ds(base, bpw)])

    return gather_kernel(user_table, movie_table, uidx2, midx2)


def _mlp_body(u_ref, m_ref, gao_ref, genres_ref, ctab_ref,
              w1_ref, b1_ref, w2_ref, b2_ref, w3_ref, b3_ref, out_ref):
    blk = u_ref.shape[0]
    lanes = lax.broadcasted_iota(jnp.int32, (blk, 128), 1)
    gao = gao_ref[...]
    g = gao[:, 0:1]
    a = gao[:, 1:2]
    o = gao[:, 2:3]
    u = u_ref[...].astype(jnp.float32)
    m = m_ref[...].astype(jnp.float32)
    oh = ((lanes == g) | (lanes == (a + 2)) | (lanes == (o + 12)))
    demo = jnp.dot(oh.astype(jnp.float32), ctab_ref[...],
                   preferred_element_type=jnp.float32)
    h1 = (
        jnp.dot(u, w1_ref[0:32, :], preferred_element_type=jnp.float32)
        + jnp.dot(m, w1_ref[32:64, :], preferred_element_type=jnp.float32)
        + jnp.dot(demo, w1_ref[64:80, :], preferred_element_type=jnp.float32)
        + jnp.dot(genres_ref[...], w1_ref[80:98, :], preferred_element_type=jnp.float32)
        + b1_ref[...]
    )
    h1 = jnp.maximum(h1, 0.0)
    h2 = jnp.maximum(
        jnp.dot(h1, w2_ref[...], preferred_element_type=jnp.float32) + b2_ref[...],
        0.0)
    out_ref[...] = (jnp.dot(h2, w3_ref[...], preferred_element_type=jnp.float32)
                    + b3_ref[...])


def _tc_mlp(u, m, gao, genres, ctab, W1, b1, W2, b2, W3, b3, blk=2048):
    grid = B // blk
    full = lambda i: (0, 0)
    return pl.pallas_call(
        _mlp_body,
        grid=(grid,),
        in_specs=[
            pl.BlockSpec((blk, EMB), lambda i: (i, 0)),
            pl.BlockSpec((blk, EMB), lambda i: (i, 0)),
            pl.BlockSpec((blk, 3), lambda i: (i, 0)),
            pl.BlockSpec((blk, 18), lambda i: (i, 0)),
            pl.BlockSpec((128, 16), full),
            pl.BlockSpec((98, 128), full),
            pl.BlockSpec((1, 128), full),
            pl.BlockSpec((128, 64), full),
            pl.BlockSpec((1, 64), full),
            pl.BlockSpec((64, 1), full),
            pl.BlockSpec((1, 1), full),
        ],
        out_specs=pl.BlockSpec((blk, 1), lambda i: (i, 0)),
        out_shape=jax.ShapeDtypeStruct((B, 1), jnp.float32),
    )(u, m, gao, genres, ctab, W1, b1, W2, b2, W3, b3)


def kernel(user, movie, gender, age, occupation, genres,
           user_table, movie_table, gender_table, age_table, occ_table,
           W1, b1, W2, b2, W3, b3):
    utb = user_table.astype(jnp.bfloat16)
    mtb = movie_table.astype(jnp.bfloat16)
    uidx2 = user.astype(jnp.int32).reshape(B // IDX_CHUNK, IDX_CHUNK)
    midx2 = movie.astype(jnp.int32).reshape(B // IDX_CHUNK, IDX_CHUNK)
    u, m = _sc_gather(utb, mtb, uidx2, midx2)

    ctab = jnp.zeros((128, 16), jnp.float32)
    ctab = ctab.at[0:2, 0:4].set(gender_table)
    ctab = ctab.at[2:12, 4:8].set(age_table)
    ctab = ctab.at[12:37, 8:16].set(occ_table)

    gao = jnp.stack([gender.astype(jnp.int32), age.astype(jnp.int32),
                     occupation.astype(jnp.int32)], axis=1)

    out = _tc_mlp(u, m, gao, genres, ctab,
                  W1, b1.reshape(1, 128), W2, b2.reshape(1, 64),
                  W3, b3.reshape(1, 1))
    return jnp.squeeze(out, axis=1)
